# baseline (speedup 1.0000x reference)
import functools

import jax
import jax.numpy as jnp
import numpy as np
from jax.experimental import pallas as pl
from jax.experimental.pallas import tpu as pltpu
from jax.experimental.pallas import tpu_sc as plsc

B = 2; S = 2048; D = 768; H = 12; DH = D // H
POOL = 8; SP = S // POOL
E = 16; DFF = 2 * D
BS = B * S
CAP = int(np.ceil(1.25 * BS / E))
NSLOT = E * CAP
NSLOT_PAD = NSLOT + 8
LBW = 0.01
ATT_SCALE = float(1.0 / np.sqrt(DH))
F32 = jnp.float32
BF16 = jnp.bfloat16


def _conv_ln_core(x, w_ref, g_ref, b_ref, xr_ref, h_ref):
    z = jnp.zeros((1, D), F32)
    w = w_ref[...]
    xp = jnp.concatenate([z, x[:-1, :]], axis=0)
    xn = jnp.concatenate([x[1:, :], z], axis=0)
    xr = x + xp * w[0:1, :] + x * w[1:2, :] + xn * w[2:3, :]
    xr_ref[0] = xr
    mu = jnp.mean(xr, axis=-1, keepdims=True)
    var = jnp.mean((xr - mu) ** 2, axis=-1, keepdims=True)
    h_ref[0] = (xr - mu) * jax.lax.rsqrt(var + 1e-5) * g_ref[...] + b_ref[...]


def _conv_ln_body(x_ref, w_ref, g_ref, b_ref, xr_ref, h_ref):
    _conv_ln_core(x_ref[0], w_ref, g_ref, b_ref, xr_ref, h_ref)


def _conv_ln(x, conv_w, g, b):
    return pl.pallas_call(
        _conv_ln_body,
        grid=(B,),
        in_specs=[
            pl.BlockSpec((1, S, D), lambda i: (i, 0, 0)),
            pl.BlockSpec((3, D), lambda i: (0, 0)),
            pl.BlockSpec((1, D), lambda i: (0, 0)),
            pl.BlockSpec((1, D), lambda i: (0, 0)),
        ],
        out_specs=[
            pl.BlockSpec((1, S, D), lambda i: (i, 0, 0)),
            pl.BlockSpec((1, S, D), lambda i: (i, 0, 0)),
        ],
        out_shape=[jax.ShapeDtypeStruct((B, S, D), F32),
                   jax.ShapeDtypeStruct((B, S, D), F32)],
    )(x, conv_w, g, b)


def _comb_conv_ln_body(x2_ref, gat_ref, kg_ref, w_ref, g_ref, b_ref,
                       xr_ref, h_ref):
    x = x2_ref[0] + gat_ref[0].astype(F32) * kg_ref[0]
    _conv_ln_core(x, w_ref, g_ref, b_ref, xr_ref, h_ref)


def _comb_conv_ln(x2, gat, kg, conv_w, g, b):
    return pl.pallas_call(
        _comb_conv_ln_body,
        grid=(B,),
        in_specs=[
            pl.BlockSpec((1, S, D), lambda i: (i, 0, 0)),
            pl.BlockSpec((1, S, D), lambda i: (i, 0, 0)),
            pl.BlockSpec((1, S, 1), lambda i: (i, 0, 0)),
            pl.BlockSpec((3, D), lambda i: (0, 0)),
            pl.BlockSpec((1, D), lambda i: (0, 0)),
            pl.BlockSpec((1, D), lambda i: (0, 0)),
        ],
        out_specs=[
            pl.BlockSpec((1, S, D), lambda i: (i, 0, 0)),
            pl.BlockSpec((1, S, D), lambda i: (i, 0, 0)),
        ],
        out_shape=[jax.ShapeDtypeStruct((B, S, D), F32),
                   jax.ShapeDtypeStruct((B, S, D), F32)],
    )(x2, gat, kg, conv_w, g, b)


def _qkv_body(h_ref, wq_ref, bq_ref, wk_ref, bk_ref, wv_ref, bv_ref,
              q_ref, k_ref, v_ref):
    h = h_ref[0]
    gi = jax.lax.broadcasted_iota(jnp.int32, (SP, S), 1)
    ci = jax.lax.broadcasted_iota(jnp.int32, (SP, S), 0)
    pm = jnp.where(gi // POOL == ci, 1.0 / POOL, 0.0).astype(F32)
    hp = jnp.dot(pm, h, preferred_element_type=F32)
    q_ref[0] = jnp.dot(h, wq_ref[...], preferred_element_type=F32) + bq_ref[...]
    k_ref[0] = jnp.dot(hp, wk_ref[...], preferred_element_type=F32) + bk_ref[...]
    v_ref[0] = jnp.dot(hp, wv_ref[...], preferred_element_type=F32) + bv_ref[...]


def _qkv(h, wq, bq, wk, bk, wv, bv):
    wspec = pl.BlockSpec((D, D), lambda i: (0, 0))
    bspec = pl.BlockSpec((1, D), lambda i: (0, 0))
    return pl.pallas_call(
        _qkv_body,
        grid=(B,),
        in_specs=[pl.BlockSpec((1, S, D), lambda i: (i, 0, 0)),
                  wspec, bspec, wspec, bspec, wspec, bspec],
        out_specs=[
            pl.BlockSpec((1, S, D), lambda i: (i, 0, 0)),
            pl.BlockSpec((1, SP, D), lambda i: (i, 0, 0)),
            pl.BlockSpec((1, SP, D), lambda i: (i, 0, 0)),
        ],
        out_shape=[
            jax.ShapeDtypeStruct((B, S, D), F32),
            jax.ShapeDtypeStruct((B, SP, D), F32),
            jax.ShapeDtypeStruct((B, SP, D), F32),
        ],
    )(h, wq, bq, wk, bk, wv, bv)


def _attn_body(q_ref, k_ref, v_ref, o_ref):
    q = q_ref[0]
    k = k_ref[0]
    v = v_ref[0]
    outs = []
    for j in range(2):
        qh = q[:, j * DH:(j + 1) * DH]
        kh = k[:, j * DH:(j + 1) * DH]
        vh = v[:, j * DH:(j + 1) * DH]
        s = jax.lax.dot_general(qh, kh, (((1,), (1,)), ((), ())),
                                preferred_element_type=F32) * ATT_SCALE
        m = jnp.max(s, axis=-1, keepdims=True)
        e = jnp.exp(s - m)
        p = e / jnp.sum(e, axis=-1, keepdims=True)
        outs.append(jnp.dot(p, vh, preferred_element_type=F32))
    o_ref[0] = jnp.concatenate(outs, axis=-1)


def _attn(q, k, v):
    return pl.pallas_call(
        _attn_body,
        grid=(B, H // 2),
        in_specs=[
            pl.BlockSpec((1, S, 2 * DH), lambda b, h: (b, 0, h)),
            pl.BlockSpec((1, SP, 2 * DH), lambda b, h: (b, 0, h)),
            pl.BlockSpec((1, SP, 2 * DH), lambda b, h: (b, 0, h)),
        ],
        out_specs=pl.BlockSpec((1, S, 2 * DH), lambda b, h: (b, 0, h)),
        out_shape=jax.ShapeDtypeStruct((B, S, D), F32),
    )(q, k, v)


def _proj_ln_body(o_ref, wo_ref, bo_ref, xr_ref, g_ref, b_ref, rw_ref,
                  x2_ref, h2_ref, lg_ref):
    x2 = xr_ref[0] + jnp.dot(o_ref[0], wo_ref[...],
                             preferred_element_type=F32) + bo_ref[...]
    x2_ref[0] = x2
    mu = jnp.mean(x2, axis=-1, keepdims=True)
    var = jnp.mean((x2 - mu) ** 2, axis=-1, keepdims=True)
    h2 = (x2 - mu) * jax.lax.rsqrt(var + 1e-5) * g_ref[...] + b_ref[...]
    h2_ref[0] = h2
    lg_ref[0] = jnp.dot(h2, rw_ref[...], preferred_element_type=F32,
                        precision=jax.lax.Precision.HIGHEST)


_PBLK = S // 2


def _proj_ln(o, wo, bo, xr, g, b, rw):
    return pl.pallas_call(
        _proj_ln_body,
        grid=(B, S // _PBLK),
        in_specs=[
            pl.BlockSpec((1, _PBLK, D), lambda i, j: (i, j, 0)),
            pl.BlockSpec((D, D), lambda i, j: (0, 0)),
            pl.BlockSpec((1, D), lambda i, j: (0, 0)),
            pl.BlockSpec((1, _PBLK, D), lambda i, j: (i, j, 0)),
            pl.BlockSpec((1, D), lambda i, j: (0, 0)),
            pl.BlockSpec((1, D), lambda i, j: (0, 0)),
            pl.BlockSpec((D, E), lambda i, j: (0, 0)),
        ],
        out_specs=[
            pl.BlockSpec((1, _PBLK, D), lambda i, j: (i, j, 0)),
            pl.BlockSpec((1, _PBLK, D), lambda i, j: (i, j, 0)),
            pl.BlockSpec((1, _PBLK, E), lambda i, j: (i, j, 0)),
        ],
        out_shape=[jax.ShapeDtypeStruct((B, S, D), F32),
                   jax.ShapeDtypeStruct((B, S, D), F32),
                   jax.ShapeDtypeStruct((B, S, E), F32)],
    )(o, wo, bo, xr, g, b, rw)


_RCH = 512


def _router_body(lg_ref, ss_ref, sg_ref, kg_ref, lb_ref, oh_scr):
    logits = lg_ref[...]
    m = jnp.max(logits, axis=-1, keepdims=True)
    ex = jnp.exp(logits - m)
    probs = ex / jnp.sum(ex, axis=-1, keepdims=True)
    gate = jnp.max(probs, axis=-1, keepdims=True)
    ii = jax.lax.broadcasted_iota(jnp.int32, (BS, E), 1)
    eidx = jnp.min(jnp.where(probs == gate, ii, E), axis=-1, keepdims=True)
    oh = (ii == eidx).astype(F32)
    oh_scr[...] = oh
    pi = jnp.mean(probs, axis=0, keepdims=True)
    fi = jnp.mean(oh, axis=0, keepdims=True)
    lb = E * jnp.sum(fi * pi)
    lb_ref[...] = jnp.full((8, 128), lb, F32)
    sg_ref[...] = eidx
    kg_ref[...] = gate
    li = jax.lax.broadcasted_iota(jnp.int32, (_RCH, _RCH), 0)
    lj = jax.lax.broadcasted_iota(jnp.int32, (_RCH, _RCH), 1)
    ltri = jnp.where(lj <= li, 1.0, 0.0).astype(F32)

    def body(c, carry):
        sl = pl.ds(c * _RCH, _RCH)
        ohc = oh_scr[sl, :]
        within = jnp.dot(ltri, ohc, preferred_element_type=F32)
        pos = jnp.sum((within + carry) * ohc, axis=-1, keepdims=True) - 1.0
        e_c = sg_ref[sl, :]
        g_c = kg_ref[sl, :]
        keep = pos < CAP
        spos = jnp.clip(pos, 0.0, CAP - 1.0).astype(jnp.int32)
        slotg = e_c * CAP + spos
        ss_ref[sl, :] = jnp.where(keep, slotg, NSLOT)
        sg_ref[sl, :] = slotg
        kg_ref[sl, :] = jnp.where(keep, g_c, 0.0)
        return carry + within[_RCH - 1:_RCH, :]

    jax.lax.fori_loop(0, BS // _RCH, body, jnp.zeros((1, E), F32))


def _router(lg):
    return pl.pallas_call(
        _router_body,
        in_specs=[pl.BlockSpec((BS, E), lambda: (0, 0))],
        out_specs=[
            pl.BlockSpec((BS, 1), lambda: (0, 0)),
            pl.BlockSpec((BS, 1), lambda: (0, 0)),
            pl.BlockSpec((BS, 1), lambda: (0, 0)),
            pl.BlockSpec((8, 128), lambda: (0, 0)),
        ],
        out_shape=[
            jax.ShapeDtypeStruct((BS, 1), jnp.int32),
            jax.ShapeDtypeStruct((BS, 1), jnp.int32),
            jax.ShapeDtypeStruct((BS, 1), F32),
            jax.ShapeDtypeStruct((8, 128), F32),
        ],
        scratch_shapes=[pltpu.VMEM((BS, E), F32)],
    )(lg)


def _expert_body(buf_ref, w1_ref, b1_ref, w2_ref, b2_ref, ob_ref):
    xb = buf_ref[...].astype(BF16)
    hdn = jnp.dot(xb, w1_ref[0].astype(BF16),
                  preferred_element_type=F32) + b1_ref[0]
    hdn = jax.nn.gelu(hdn).astype(BF16)
    ob_ref[...] = jnp.dot(hdn, w2_ref[0].astype(BF16),
                          preferred_element_type=F32) + b2_ref[0]


def _expert_ffn(buf, w1, b1r, w2, b2r):
    return pl.pallas_call(
        _expert_body,
        grid=(E,),
        in_specs=[
            pl.BlockSpec((CAP, D), lambda i: (i, 0)),
            pl.BlockSpec((1, D, DFF), lambda i: (i, 0, 0)),
            pl.BlockSpec((1, 1, DFF), lambda i: (i, 0, 0)),
            pl.BlockSpec((1, DFF, D), lambda i: (i, 0, 0)),
            pl.BlockSpec((1, 1, D), lambda i: (i, 0, 0)),
        ],
        out_specs=pl.BlockSpec((CAP, D), lambda i: (i, 0)),
        out_shape=jax.ShapeDtypeStruct((NSLOT, D), F32),
    )(buf, w1, b1r, w2, b2r)


_CCH = 512


def _combine_body(x2_ref, g_ref, kg_ref, x3_ref):
    x3_ref[...] = x2_ref[...] + g_ref[...].astype(F32) * kg_ref[...]


def _combine(x2f, gat, kg):
    return pl.pallas_call(
        _combine_body,
        grid=(BS // _CCH,),
        in_specs=[
            pl.BlockSpec((_CCH, D), lambda i: (i, 0)),
            pl.BlockSpec((_CCH, D), lambda i: (i, 0)),
            pl.BlockSpec((_CCH, 1), lambda i: (i, 0)),
        ],
        out_specs=pl.BlockSpec((_CCH, D), lambda i: (i, 0)),
        out_shape=jax.ShapeDtypeStruct((BS, D), F32),
    )(x2f, gat, kg)


_NC = 2
_NS = 16
_NW = _NC * _NS
_TPW = BS // _NW


def _sc_mesh():
    return plsc.VectorSubcoreMesh(core_axis_name="c", subcore_axis_name="s")


def _sc_dispatch(xf, slots):
    @functools.partial(
        pl.kernel,
        mesh=_sc_mesh(),
        out_type=jax.ShapeDtypeStruct((NSLOT_PAD, D), F32),
        scratch_types=[
            pltpu.VMEM((_TPW,), jnp.int32),
            pltpu.VMEM((_TPW, D), F32),
            pltpu.SemaphoreType.DMA,
        ],
    )
    def k(x_hbm, i_hbm, o_hbm, idx_v, rows_v, sem):
        wid = jax.lax.axis_index("s") * _NC + jax.lax.axis_index("c")
        base = wid * _TPW
        pltpu.sync_copy(i_hbm.at[pl.ds(base, _TPW)], idx_v)
        pltpu.sync_copy(x_hbm.at[pl.ds(base, _TPW)], rows_v)
        pltpu.async_copy(rows_v, o_hbm.at[idx_v], sem).wait()

    return k(xf, slots)


def _sc_combine(ob, slots):
    @functools.partial(
        pl.kernel,
        mesh=_sc_mesh(),
        out_type=jax.ShapeDtypeStruct((BS, D), F32),
        scratch_types=[
            pltpu.VMEM((_TPW,), jnp.int32),
            pltpu.VMEM((_TPW, D), F32),
            pltpu.SemaphoreType.DMA,
        ],
    )
    def k(t_hbm, i_hbm, o_hbm, idx_v, rows_v, sem):
        wid = jax.lax.axis_index("s") * _NC + jax.lax.axis_index("c")
        base = wid * _TPW
        pltpu.sync_copy(i_hbm.at[pl.ds(base, _TPW)], idx_v)
        pltpu.async_copy(t_hbm.at[idx_v], rows_v, sem).wait()
        pltpu.sync_copy(rows_v, o_hbm.at[pl.ds(base, _TPW)])

    return k(ob, slots)


def kernel(x, ln1_g, ln1_b, ln2_g, ln2_b, conv_w, Wq, bq, Wk, bk, Wv, bv,
           Wo, bo, router_w, W1, b1, W2, b2):
    ln1g = ln1_g.reshape(1, D)
    ln1b = ln1_b.reshape(1, D)
    ln2g = ln2_g.reshape(1, D)
    ln2b = ln2_b.reshape(1, D)
    bq2 = bq.reshape(1, D)
    bk2 = bk.reshape(1, D)
    bv2 = bv.reshape(1, D)
    bo2 = bo.reshape(1, D)
    b1r = b1.reshape(E, 1, DFF)
    b2r = b2.reshape(E, 1, D)
    aux = jnp.zeros((), F32)
    for li in range(2):
        if li == 0:
            xr, h = _conv_ln(x, conv_w, ln1g, ln1b)
        else:
            xr, h = _comb_conv_ln(x2, gat.reshape(B, S, D),
                                  kg.reshape(B, S, 1), conv_w, ln1g, ln1b)
        q, k, v = _qkv(h, Wq, bq2, Wk, bk2, Wv, bv2)
        o = _attn(q, k, v)
        x2, h2, lg = _proj_ln(o, Wo, bo2, xr, ln2g, ln2b, router_w)
        h2f = h2.reshape(BS, D)
        ss, sg, kg, lbarr = _router(lg.reshape(BS, E))
        buf = _sc_dispatch(h2f, ss.reshape(BS))
        ob = _expert_ffn(buf, W1, b1r, W2, b2r)
        gat = _sc_combine(ob, sg.reshape(BS))
        aux = aux + LBW * lbarr[0, 0]
    x3f = _combine(x2.reshape(BS, D), gat, kg)
    return x3f.reshape(B, S, D), aux


# speedup vs baseline: 1.1104x; 1.1104x over previous
import functools

import jax
import jax.numpy as jnp
import numpy as np
from jax.experimental import pallas as pl
from jax.experimental.pallas import tpu as pltpu
from jax.experimental.pallas import tpu_sc as plsc

B = 2; S = 2048; D = 768; H = 12; DH = D // H
POOL = 8; SP = S // POOL
E = 16; DFF = 2 * D
BS = B * S
CAP = int(np.ceil(1.25 * BS / E))
NSLOT = E * CAP
NSLOT_PAD = NSLOT + 8
LBW = 0.01
ATT_SCALE = float(1.0 / np.sqrt(DH))
F32 = jnp.float32
BF16 = jnp.bfloat16


def _conv_ln_core(x, w_ref, g_ref, b_ref, xr_ref, h_ref):
    z = jnp.zeros((1, D), F32)
    w = w_ref[...]
    xp = jnp.concatenate([z, x[:-1, :]], axis=0)
    xn = jnp.concatenate([x[1:, :], z], axis=0)
    xr = x + xp * w[0:1, :] + x * w[1:2, :] + xn * w[2:3, :]
    xr_ref[0] = xr
    mu = jnp.mean(xr, axis=-1, keepdims=True)
    var = jnp.mean((xr - mu) ** 2, axis=-1, keepdims=True)
    h_ref[0] = (xr - mu) * jax.lax.rsqrt(var + 1e-5) * g_ref[...] + b_ref[...]


def _conv_ln_body(x_ref, w_ref, g_ref, b_ref, xr_ref, h_ref):
    _conv_ln_core(x_ref[0], w_ref, g_ref, b_ref, xr_ref, h_ref)


def _conv_ln(x, conv_w, g, b):
    return pl.pallas_call(
        _conv_ln_body,
        grid=(B,),
        in_specs=[
            pl.BlockSpec((1, S, D), lambda i: (i, 0, 0)),
            pl.BlockSpec((3, D), lambda i: (0, 0)),
            pl.BlockSpec((1, D), lambda i: (0, 0)),
            pl.BlockSpec((1, D), lambda i: (0, 0)),
        ],
        out_specs=[
            pl.BlockSpec((1, S, D), lambda i: (i, 0, 0)),
            pl.BlockSpec((1, S, D), lambda i: (i, 0, 0)),
        ],
        out_shape=[jax.ShapeDtypeStruct((B, S, D), F32),
                   jax.ShapeDtypeStruct((B, S, D), F32)],
    )(x, conv_w, g, b)


def _comb_conv_ln_body(x2_ref, gat_ref, kg_ref, w_ref, g_ref, b_ref,
                       xr_ref, h_ref):
    x = x2_ref[0] + gat_ref[0].astype(F32) * kg_ref[0]
    _conv_ln_core(x, w_ref, g_ref, b_ref, xr_ref, h_ref)


def _comb_conv_ln(x2, gat, kg, conv_w, g, b):
    return pl.pallas_call(
        _comb_conv_ln_body,
        grid=(B,),
        in_specs=[
            pl.BlockSpec((1, S, D), lambda i: (i, 0, 0)),
            pl.BlockSpec((1, S, D), lambda i: (i, 0, 0)),
            pl.BlockSpec((1, S, 1), lambda i: (i, 0, 0)),
            pl.BlockSpec((3, D), lambda i: (0, 0)),
            pl.BlockSpec((1, D), lambda i: (0, 0)),
            pl.BlockSpec((1, D), lambda i: (0, 0)),
        ],
        out_specs=[
            pl.BlockSpec((1, S, D), lambda i: (i, 0, 0)),
            pl.BlockSpec((1, S, D), lambda i: (i, 0, 0)),
        ],
        out_shape=[jax.ShapeDtypeStruct((B, S, D), F32),
                   jax.ShapeDtypeStruct((B, S, D), F32)],
    )(x2, gat, kg, conv_w, g, b)


def _qkv_body(h_ref, wq_ref, bq_ref, wk_ref, bk_ref, wv_ref, bv_ref,
              q_ref, k_ref, v_ref):
    h = h_ref[0]
    gi = jax.lax.broadcasted_iota(jnp.int32, (SP, S), 1)
    ci = jax.lax.broadcasted_iota(jnp.int32, (SP, S), 0)
    pm = jnp.where(gi // POOL == ci, 1.0 / POOL, 0.0).astype(F32)
    hp = jnp.dot(pm, h, preferred_element_type=F32)
    hb = h.astype(BF16)
    hpb = hp.astype(BF16)
    q = jnp.dot(hb, wq_ref[...].astype(BF16), preferred_element_type=F32) + bq_ref[...]
    k = jnp.dot(hpb, wk_ref[...].astype(BF16), preferred_element_type=F32) + bk_ref[...]
    v = jnp.dot(hpb, wv_ref[...].astype(BF16), preferred_element_type=F32) + bv_ref[...]
    q_ref[0] = q.astype(BF16)
    k_ref[0] = k.astype(BF16)
    v_ref[0] = v.astype(BF16)


def _qkv(h, wq, bq, wk, bk, wv, bv):
    wspec = pl.BlockSpec((D, D), lambda i: (0, 0))
    bspec = pl.BlockSpec((1, D), lambda i: (0, 0))
    return pl.pallas_call(
        _qkv_body,
        grid=(B,),
        in_specs=[pl.BlockSpec((1, S, D), lambda i: (i, 0, 0)),
                  wspec, bspec, wspec, bspec, wspec, bspec],
        out_specs=[
            pl.BlockSpec((1, S, D), lambda i: (i, 0, 0)),
            pl.BlockSpec((1, SP, D), lambda i: (i, 0, 0)),
            pl.BlockSpec((1, SP, D), lambda i: (i, 0, 0)),
        ],
        out_shape=[
            jax.ShapeDtypeStruct((B, S, D), BF16),
            jax.ShapeDtypeStruct((B, SP, D), BF16),
            jax.ShapeDtypeStruct((B, SP, D), BF16),
        ],
    )(h, wq, bq, wk, bk, wv, bv)


def _attn_body(q_ref, k_ref, v_ref, o_ref):
    q = q_ref[0]
    k = k_ref[0]
    v = v_ref[0]
    outs = []
    for j in range(2):
        qh = q[:, j * DH:(j + 1) * DH]
        kh = k[:, j * DH:(j + 1) * DH]
        vh = v[:, j * DH:(j + 1) * DH]
        s = jax.lax.dot_general(qh, kh, (((1,), (1,)), ((), ())),
                                preferred_element_type=F32) * ATT_SCALE
        m = jnp.max(s, axis=-1, keepdims=True)
        e = jnp.exp(s - m)
        p = (e / jnp.sum(e, axis=-1, keepdims=True)).astype(BF16)
        outs.append(jnp.dot(p, vh, preferred_element_type=F32))
    o_ref[0] = jnp.concatenate(outs, axis=-1).astype(BF16)


def _attn(q, k, v):
    return pl.pallas_call(
        _attn_body,
        grid=(B, H // 2),
        in_specs=[
            pl.BlockSpec((1, S, 2 * DH), lambda b, h: (b, 0, h)),
            pl.BlockSpec((1, SP, 2 * DH), lambda b, h: (b, 0, h)),
            pl.BlockSpec((1, SP, 2 * DH), lambda b, h: (b, 0, h)),
        ],
        out_specs=pl.BlockSpec((1, S, 2 * DH), lambda b, h: (b, 0, h)),
        out_shape=jax.ShapeDtypeStruct((B, S, D), BF16),
    )(q, k, v)


def _proj_ln_body(o_ref, wo_ref, bo_ref, xr_ref, g_ref, b_ref, rw_ref,
                  x2_ref, h2_ref, lg_ref):
    x2 = xr_ref[0] + jnp.dot(o_ref[0], wo_ref[...].astype(BF16),
                             preferred_element_type=F32) + bo_ref[...]
    x2_ref[0] = x2
    mu = jnp.mean(x2, axis=-1, keepdims=True)
    var = jnp.mean((x2 - mu) ** 2, axis=-1, keepdims=True)
    h2 = (x2 - mu) * jax.lax.rsqrt(var + 1e-5) * g_ref[...] + b_ref[...]
    h2_ref[0] = h2
    lg_ref[0] = jnp.dot(h2, rw_ref[...], preferred_element_type=F32)


_PBLK = S // 2


def _proj_ln(o, wo, bo, xr, g, b, rw):
    return pl.pallas_call(
        _proj_ln_body,
        grid=(B, S // _PBLK),
        in_specs=[
            pl.BlockSpec((1, _PBLK, D), lambda i, j: (i, j, 0)),
            pl.BlockSpec((D, D), lambda i, j: (0, 0)),
            pl.BlockSpec((1, D), lambda i, j: (0, 0)),
            pl.BlockSpec((1, _PBLK, D), lambda i, j: (i, j, 0)),
            pl.BlockSpec((1, D), lambda i, j: (0, 0)),
            pl.BlockSpec((1, D), lambda i, j: (0, 0)),
            pl.BlockSpec((D, E), lambda i, j: (0, 0)),
        ],
        out_specs=[
            pl.BlockSpec((1, _PBLK, D), lambda i, j: (i, j, 0)),
            pl.BlockSpec((1, _PBLK, D), lambda i, j: (i, j, 0)),
            pl.BlockSpec((1, _PBLK, E), lambda i, j: (i, j, 0)),
        ],
        out_shape=[jax.ShapeDtypeStruct((B, S, D), F32),
                   jax.ShapeDtypeStruct((B, S, D), F32),
                   jax.ShapeDtypeStruct((B, S, E), F32)],
    )(o, wo, bo, xr, g, b, rw)


_RCH = 512


def _router_body(lg_ref, ss_ref, sg_ref, kg_ref, lb_ref, oh_scr):
    logits = lg_ref[...]
    m = jnp.max(logits, axis=-1, keepdims=True)
    ex = jnp.exp(logits - m)
    probs = ex / jnp.sum(ex, axis=-1, keepdims=True)
    gate = jnp.max(probs, axis=-1, keepdims=True)
    ii = jax.lax.broadcasted_iota(jnp.int32, (BS, E), 1)
    eidx = jnp.min(jnp.where(probs == gate, ii, E), axis=-1, keepdims=True)
    oh = (ii == eidx).astype(F32)
    oh_scr[...] = oh
    pi = jnp.mean(probs, axis=0, keepdims=True)
    fi = jnp.mean(oh, axis=0, keepdims=True)
    lb = E * jnp.sum(fi * pi)
    lb_ref[...] = jnp.full((8, 128), lb, F32)
    sg_ref[...] = eidx
    kg_ref[...] = gate
    li = jax.lax.broadcasted_iota(jnp.int32, (_RCH, _RCH), 0)
    lj = jax.lax.broadcasted_iota(jnp.int32, (_RCH, _RCH), 1)
    ltri = jnp.where(lj <= li, 1.0, 0.0).astype(F32)

    def body(c, carry):
        sl = pl.ds(c * _RCH, _RCH)
        ohc = oh_scr[sl, :]
        within = jnp.dot(ltri, ohc, preferred_element_type=F32)
        pos = jnp.sum((within + carry) * ohc, axis=-1, keepdims=True) - 1.0
        e_c = sg_ref[sl, :]
        g_c = kg_ref[sl, :]
        keep = pos < CAP
        spos = jnp.clip(pos, 0.0, CAP - 1.0).astype(jnp.int32)
        slotg = e_c * CAP + spos
        ss_ref[sl, :] = jnp.where(keep, slotg, NSLOT)
        sg_ref[sl, :] = slotg
        kg_ref[sl, :] = jnp.where(keep, g_c, 0.0)
        return carry + within[_RCH - 1:_RCH, :]

    jax.lax.fori_loop(0, BS // _RCH, body, jnp.zeros((1, E), F32))


def _router(lg):
    return pl.pallas_call(
        _router_body,
        in_specs=[pl.BlockSpec((BS, E), lambda: (0, 0))],
        out_specs=[
            pl.BlockSpec((BS, 1), lambda: (0, 0)),
            pl.BlockSpec((BS, 1), lambda: (0, 0)),
            pl.BlockSpec((BS, 1), lambda: (0, 0)),
            pl.BlockSpec((8, 128), lambda: (0, 0)),
        ],
        out_shape=[
            jax.ShapeDtypeStruct((BS, 1), jnp.int32),
            jax.ShapeDtypeStruct((BS, 1), jnp.int32),
            jax.ShapeDtypeStruct((BS, 1), F32),
            jax.ShapeDtypeStruct((8, 128), F32),
        ],
        scratch_shapes=[pltpu.VMEM((BS, E), F32)],
    )(lg)


def _expert_body(buf_ref, w1_ref, b1_ref, w2_ref, b2_ref, ob_ref):
    xb = buf_ref[...].astype(BF16)
    hdn = jnp.dot(xb, w1_ref[0].astype(BF16),
                  preferred_element_type=F32) + b1_ref[0]
    hdn = jax.nn.gelu(hdn).astype(BF16)
    ob_ref[...] = jnp.dot(hdn, w2_ref[0].astype(BF16),
                          preferred_element_type=F32) + b2_ref[0]


def _expert_ffn(buf, w1, b1r, w2, b2r):
    return pl.pallas_call(
        _expert_body,
        grid=(E,),
        in_specs=[
            pl.BlockSpec((CAP, D), lambda i: (i, 0)),
            pl.BlockSpec((1, D, DFF), lambda i: (i, 0, 0)),
            pl.BlockSpec((1, 1, DFF), lambda i: (i, 0, 0)),
            pl.BlockSpec((1, DFF, D), lambda i: (i, 0, 0)),
            pl.BlockSpec((1, 1, D), lambda i: (i, 0, 0)),
        ],
        out_specs=pl.BlockSpec((CAP, D), lambda i: (i, 0)),
        out_shape=jax.ShapeDtypeStruct((NSLOT, D), F32),
    )(buf, w1, b1r, w2, b2r)


_CCH = 512


def _combine_body(x2_ref, g_ref, kg_ref, x3_ref):
    x3_ref[...] = x2_ref[...] + g_ref[...].astype(F32) * kg_ref[...]


def _combine(x2f, gat, kg):
    return pl.pallas_call(
        _combine_body,
        grid=(BS // _CCH,),
        in_specs=[
            pl.BlockSpec((_CCH, D), lambda i: (i, 0)),
            pl.BlockSpec((_CCH, D), lambda i: (i, 0)),
            pl.BlockSpec((_CCH, 1), lambda i: (i, 0)),
        ],
        out_specs=pl.BlockSpec((_CCH, D), lambda i: (i, 0)),
        out_shape=jax.ShapeDtypeStruct((BS, D), F32),
    )(x2f, gat, kg)


_NC = 2
_NS = 16
_NW = _NC * _NS
_TPW = BS // _NW


def _sc_mesh():
    return plsc.VectorSubcoreMesh(core_axis_name="c", subcore_axis_name="s")


def _sc_dispatch(xf, slots):
    @functools.partial(
        pl.kernel,
        mesh=_sc_mesh(),
        out_type=jax.ShapeDtypeStruct((NSLOT_PAD, D), F32),
        scratch_types=[
            pltpu.VMEM((_TPW,), jnp.int32),
            pltpu.VMEM((_TPW, D), F32),
            pltpu.SemaphoreType.DMA,
        ],
    )
    def k(x_hbm, i_hbm, o_hbm, idx_v, rows_v, sem):
        wid = jax.lax.axis_index("s") * _NC + jax.lax.axis_index("c")
        base = wid * _TPW
        pltpu.sync_copy(i_hbm.at[pl.ds(base, _TPW)], idx_v)
        pltpu.sync_copy(x_hbm.at[pl.ds(base, _TPW)], rows_v)
        pltpu.async_copy(rows_v, o_hbm.at[idx_v], sem).wait()

    return k(xf, slots)


def _sc_combine(ob, slots):
    @functools.partial(
        pl.kernel,
        mesh=_sc_mesh(),
        out_type=jax.ShapeDtypeStruct((BS, D), F32),
        scratch_types=[
            pltpu.VMEM((_TPW,), jnp.int32),
            pltpu.VMEM((_TPW, D), F32),
            pltpu.SemaphoreType.DMA,
        ],
    )
    def k(t_hbm, i_hbm, o_hbm, idx_v, rows_v, sem):
        wid = jax.lax.axis_index("s") * _NC + jax.lax.axis_index("c")
        base = wid * _TPW
        pltpu.sync_copy(i_hbm.at[pl.ds(base, _TPW)], idx_v)
        pltpu.async_copy(t_hbm.at[idx_v], rows_v, sem).wait()
        pltpu.sync_copy(rows_v, o_hbm.at[pl.ds(base, _TPW)])

    return k(ob, slots)


def kernel(x, ln1_g, ln1_b, ln2_g, ln2_b, conv_w, Wq, bq, Wk, bk, Wv, bv,
           Wo, bo, router_w, W1, b1, W2, b2):
    ln1g = ln1_g.reshape(1, D)
    ln1b = ln1_b.reshape(1, D)
    ln2g = ln2_g.reshape(1, D)
    ln2b = ln2_b.reshape(1, D)
    bq2 = bq.reshape(1, D)
    bk2 = bk.reshape(1, D)
    bv2 = bv.reshape(1, D)
    bo2 = bo.reshape(1, D)
    b1r = b1.reshape(E, 1, DFF)
    b2r = b2.reshape(E, 1, D)
    aux = jnp.zeros((), F32)
    for li in range(2):
        if li == 0:
            xr, h = _conv_ln(x, conv_w, ln1g, ln1b)
        else:
            xr, h = _comb_conv_ln(x2, gat.reshape(B, S, D),
                                  kg.reshape(B, S, 1), conv_w, ln1g, ln1b)
        q, k, v = _qkv(h, Wq, bq2, Wk, bk2, Wv, bv2)
        o = _attn(q, k, v)
        x2, h2, lg = _proj_ln(o, Wo, bo2, xr, ln2g, ln2b, router_w)
        h2f = h2.reshape(BS, D)
        ss, sg, kg, lbarr = _router(lg.reshape(BS, E))
        buf = _sc_dispatch(h2f, ss.reshape(BS))
        ob = _expert_ffn(buf, W1, b1r, W2, b2r)
        gat = _sc_combine(ob, sg.reshape(BS))
        aux = aux + LBW * lbarr[0, 0]
    x3f = _combine(x2.reshape(BS, D), gat, kg)
    return x3f.reshape(B, S, D), aux


# speedup vs baseline: 1.1151x; 1.0042x over previous
import functools

import jax
import jax.numpy as jnp
import numpy as np
from jax.experimental import pallas as pl
from jax.experimental.pallas import tpu as pltpu
from jax.experimental.pallas import tpu_sc as plsc

B = 2; S = 2048; D = 768; H = 12; DH = D // H
POOL = 8; SP = S // POOL
E = 16; DFF = 2 * D
BS = B * S
CAP = int(np.ceil(1.25 * BS / E))
NSLOT = E * CAP
NSLOT_PAD = NSLOT + 8
LBW = 0.01
ATT_SCALE = float(1.0 / np.sqrt(DH))
F32 = jnp.float32
BF16 = jnp.bfloat16


def _conv_ln_core(x, w_ref, g_ref, b_ref, xr_ref, h_ref):
    z = jnp.zeros((1, D), F32)
    w = w_ref[...]
    xp = jnp.concatenate([z, x[:-1, :]], axis=0)
    xn = jnp.concatenate([x[1:, :], z], axis=0)
    xr = x + xp * w[0:1, :] + x * w[1:2, :] + xn * w[2:3, :]
    xr_ref[0] = xr
    mu = jnp.mean(xr, axis=-1, keepdims=True)
    var = jnp.mean((xr - mu) ** 2, axis=-1, keepdims=True)
    h = (xr - mu) * jax.lax.rsqrt(var + 1e-5) * g_ref[...] + b_ref[...]
    h_ref[0] = h.astype(BF16)


def _conv_ln_body(x_ref, w_ref, g_ref, b_ref, xr_ref, h_ref):
    _conv_ln_core(x_ref[0], w_ref, g_ref, b_ref, xr_ref, h_ref)


def _conv_ln(x, conv_w, g, b):
    return pl.pallas_call(
        _conv_ln_body,
        grid=(B,),
        in_specs=[
            pl.BlockSpec((1, S, D), lambda i: (i, 0, 0)),
            pl.BlockSpec((3, D), lambda i: (0, 0)),
            pl.BlockSpec((1, D), lambda i: (0, 0)),
            pl.BlockSpec((1, D), lambda i: (0, 0)),
        ],
        out_specs=[
            pl.BlockSpec((1, S, D), lambda i: (i, 0, 0)),
            pl.BlockSpec((1, S, D), lambda i: (i, 0, 0)),
        ],
        out_shape=[jax.ShapeDtypeStruct((B, S, D), F32),
                   jax.ShapeDtypeStruct((B, S, D), BF16)],
    )(x, conv_w, g, b)


def _comb_conv_ln_body(x2_ref, gat_ref, kg_ref, w_ref, g_ref, b_ref,
                       xr_ref, h_ref):
    x = x2_ref[0] + gat_ref[0].astype(F32) * kg_ref[0]
    _conv_ln_core(x, w_ref, g_ref, b_ref, xr_ref, h_ref)


def _comb_conv_ln(x2, gat, kg, conv_w, g, b):
    return pl.pallas_call(
        _comb_conv_ln_body,
        grid=(B,),
        in_specs=[
            pl.BlockSpec((1, S, D), lambda i: (i, 0, 0)),
            pl.BlockSpec((1, S, D), lambda i: (i, 0, 0)),
            pl.BlockSpec((1, S, 1), lambda i: (i, 0, 0)),
            pl.BlockSpec((3, D), lambda i: (0, 0)),
            pl.BlockSpec((1, D), lambda i: (0, 0)),
            pl.BlockSpec((1, D), lambda i: (0, 0)),
        ],
        out_specs=[
            pl.BlockSpec((1, S, D), lambda i: (i, 0, 0)),
            pl.BlockSpec((1, S, D), lambda i: (i, 0, 0)),
        ],
        out_shape=[jax.ShapeDtypeStruct((B, S, D), F32),
                   jax.ShapeDtypeStruct((B, S, D), BF16)],
    )(x2, gat, kg, conv_w, g, b)


def _qkv_attn_body(h_ref, wq_ref, bq_ref, wk_ref, bk_ref, wv_ref, bv_ref,
                   o_ref):
    hb = h_ref[0]
    gi = jax.lax.broadcasted_iota(jnp.int32, (SP, S), 1)
    ci = jax.lax.broadcasted_iota(jnp.int32, (SP, S), 0)
    pm = jnp.where(gi // POOL == ci, 1.0 / POOL, 0.0).astype(BF16)
    hpb = jnp.dot(pm, hb, preferred_element_type=F32).astype(BF16)
    q = (jnp.dot(hb, wq_ref[...].astype(BF16), preferred_element_type=F32)
         + bq_ref[...]).astype(BF16)
    k = (jnp.dot(hpb, wk_ref[...].astype(BF16), preferred_element_type=F32)
         + bk_ref[...]).astype(BF16)
    v = (jnp.dot(hpb, wv_ref[...].astype(BF16), preferred_element_type=F32)
         + bv_ref[...]).astype(BF16)
    outs = []
    for j in range(H):
        qh = q[:, j * DH:(j + 1) * DH]
        kh = k[:, j * DH:(j + 1) * DH]
        vh = v[:, j * DH:(j + 1) * DH]
        s = jax.lax.dot_general(qh, kh, (((1,), (1,)), ((), ())),
                                preferred_element_type=F32) * ATT_SCALE
        m = jnp.max(s, axis=-1, keepdims=True)
        e = jnp.exp(s - m)
        p = (e / jnp.sum(e, axis=-1, keepdims=True)).astype(BF16)
        outs.append(jnp.dot(p, vh, preferred_element_type=F32))
    o_ref[0] = jnp.concatenate(outs, axis=-1).astype(BF16)


def _qkv_attn(h, wq, bq, wk, bk, wv, bv):
    wspec = pl.BlockSpec((D, D), lambda i: (0, 0))
    bspec = pl.BlockSpec((1, D), lambda i: (0, 0))
    return pl.pallas_call(
        _qkv_attn_body,
        grid=(B,),
        in_specs=[pl.BlockSpec((1, S, D), lambda i: (i, 0, 0)),
                  wspec, bspec, wspec, bspec, wspec, bspec],
        out_specs=pl.BlockSpec((1, S, D), lambda i: (i, 0, 0)),
        out_shape=jax.ShapeDtypeStruct((B, S, D), BF16),
    )(h, wq, bq, wk, bk, wv, bv)


def _proj_ln_body(o_ref, wo_ref, bo_ref, xr_ref, g_ref, b_ref, rw_ref,
                  x2_ref, h2_ref, lg_ref):
    x2 = xr_ref[0] + jnp.dot(o_ref[0], wo_ref[...].astype(BF16),
                             preferred_element_type=F32) + bo_ref[...]
    x2_ref[0] = x2
    mu = jnp.mean(x2, axis=-1, keepdims=True)
    var = jnp.mean((x2 - mu) ** 2, axis=-1, keepdims=True)
    h2 = (x2 - mu) * jax.lax.rsqrt(var + 1e-5) * g_ref[...] + b_ref[...]
    h2_ref[0] = h2
    lg_ref[0] = jnp.dot(h2, rw_ref[...], preferred_element_type=F32)


_PBLK = S // 2


def _proj_ln(o, wo, bo, xr, g, b, rw):
    return pl.pallas_call(
        _proj_ln_body,
        grid=(B, S // _PBLK),
        in_specs=[
            pl.BlockSpec((1, _PBLK, D), lambda i, j: (i, j, 0)),
            pl.BlockSpec((D, D), lambda i, j: (0, 0)),
            pl.BlockSpec((1, D), lambda i, j: (0, 0)),
            pl.BlockSpec((1, _PBLK, D), lambda i, j: (i, j, 0)),
            pl.BlockSpec((1, D), lambda i, j: (0, 0)),
            pl.BlockSpec((1, D), lambda i, j: (0, 0)),
            pl.BlockSpec((D, E), lambda i, j: (0, 0)),
        ],
        out_specs=[
            pl.BlockSpec((1, _PBLK, D), lambda i, j: (i, j, 0)),
            pl.BlockSpec((1, _PBLK, D), lambda i, j: (i, j, 0)),
            pl.BlockSpec((1, _PBLK, E), lambda i, j: (i, j, 0)),
        ],
        out_shape=[jax.ShapeDtypeStruct((B, S, D), F32),
                   jax.ShapeDtypeStruct((B, S, D), F32),
                   jax.ShapeDtypeStruct((B, S, E), F32)],
    )(o, wo, bo, xr, g, b, rw)


_RCH = 512


def _router_body(lg_ref, ss_ref, sg_ref, kg_ref, lb_ref, oh_scr):
    logits = lg_ref[...]
    m = jnp.max(logits, axis=-1, keepdims=True)
    ex = jnp.exp(logits - m)
    probs = ex / jnp.sum(ex, axis=-1, keepdims=True)
    gate = jnp.max(probs, axis=-1, keepdims=True)
    ii = jax.lax.broadcasted_iota(jnp.int32, (BS, E), 1)
    eidx = jnp.min(jnp.where(probs == gate, ii, E), axis=-1, keepdims=True)
    oh = (ii == eidx).astype(F32)
    oh_scr[...] = oh
    pi = jnp.mean(probs, axis=0, keepdims=True)
    fi = jnp.mean(oh, axis=0, keepdims=True)
    lb = E * jnp.sum(fi * pi)
    lb_ref[...] = jnp.full((8, 128), lb, F32)
    sg_ref[...] = eidx
    kg_ref[...] = gate
    li = jax.lax.broadcasted_iota(jnp.int32, (_RCH, _RCH), 0)
    lj = jax.lax.broadcasted_iota(jnp.int32, (_RCH, _RCH), 1)
    ltri = jnp.where(lj <= li, 1.0, 0.0).astype(F32)

    def body(c, carry):
        sl = pl.ds(c * _RCH, _RCH)
        ohc = oh_scr[sl, :]
        within = jnp.dot(ltri, ohc, preferred_element_type=F32)
        pos = jnp.sum((within + carry) * ohc, axis=-1, keepdims=True) - 1.0
        e_c = sg_ref[sl, :]
        g_c = kg_ref[sl, :]
        keep = pos < CAP
        spos = jnp.clip(pos, 0.0, CAP - 1.0).astype(jnp.int32)
        slotg = e_c * CAP + spos
        ss_ref[sl, :] = jnp.where(keep, slotg, NSLOT)
        sg_ref[sl, :] = slotg
        kg_ref[sl, :] = jnp.where(keep, g_c, 0.0)
        return carry + within[_RCH - 1:_RCH, :]

    jax.lax.fori_loop(0, BS // _RCH, body, jnp.zeros((1, E), F32))


def _router(lg):
    return pl.pallas_call(
        _router_body,
        in_specs=[pl.BlockSpec((BS, E), lambda: (0, 0))],
        out_specs=[
            pl.BlockSpec((BS, 1), lambda: (0, 0)),
            pl.BlockSpec((BS, 1), lambda: (0, 0)),
            pl.BlockSpec((BS, 1), lambda: (0, 0)),
            pl.BlockSpec((8, 128), lambda: (0, 0)),
        ],
        out_shape=[
            jax.ShapeDtypeStruct((BS, 1), jnp.int32),
            jax.ShapeDtypeStruct((BS, 1), jnp.int32),
            jax.ShapeDtypeStruct((BS, 1), F32),
            jax.ShapeDtypeStruct((8, 128), F32),
        ],
        scratch_shapes=[pltpu.VMEM((BS, E), F32)],
    )(lg)


def _expert_body(buf_ref, w1_ref, b1_ref, w2_ref, b2_ref, ob_ref):
    xb = buf_ref[...].astype(BF16)
    hdn = jnp.dot(xb, w1_ref[0].astype(BF16),
                  preferred_element_type=F32) + b1_ref[0]
    hdn = jax.nn.gelu(hdn).astype(BF16)
    ob_ref[...] = jnp.dot(hdn, w2_ref[0].astype(BF16),
                          preferred_element_type=F32) + b2_ref[0]


def _expert_ffn(buf, w1, b1r, w2, b2r):
    return pl.pallas_call(
        _expert_body,
        grid=(E,),
        in_specs=[
            pl.BlockSpec((CAP, D), lambda i: (i, 0)),
            pl.BlockSpec((1, D, DFF), lambda i: (i, 0, 0)),
            pl.BlockSpec((1, 1, DFF), lambda i: (i, 0, 0)),
            pl.BlockSpec((1, DFF, D), lambda i: (i, 0, 0)),
            pl.BlockSpec((1, 1, D), lambda i: (i, 0, 0)),
        ],
        out_specs=pl.BlockSpec((CAP, D), lambda i: (i, 0)),
        out_shape=jax.ShapeDtypeStruct((NSLOT, D), F32),
    )(buf, w1, b1r, w2, b2r)


_CCH = 512


def _combine_body(x2_ref, g_ref, kg_ref, x3_ref):
    x3_ref[...] = x2_ref[...] + g_ref[...].astype(F32) * kg_ref[...]


def _combine(x2f, gat, kg):
    return pl.pallas_call(
        _combine_body,
        grid=(BS // _CCH,),
        in_specs=[
            pl.BlockSpec((_CCH, D), lambda i: (i, 0)),
            pl.BlockSpec((_CCH, D), lambda i: (i, 0)),
            pl.BlockSpec((_CCH, 1), lambda i: (i, 0)),
        ],
        out_specs=pl.BlockSpec((_CCH, D), lambda i: (i, 0)),
        out_shape=jax.ShapeDtypeStruct((BS, D), F32),
    )(x2f, gat, kg)


_NC = 2
_NS = 16
_NW = _NC * _NS
_TPW = BS // _NW


def _sc_mesh():
    return plsc.VectorSubcoreMesh(core_axis_name="c", subcore_axis_name="s")


def _sc_dispatch(xf, slots):
    @functools.partial(
        pl.kernel,
        mesh=_sc_mesh(),
        out_type=jax.ShapeDtypeStruct((NSLOT_PAD, D), F32),
        scratch_types=[
            pltpu.VMEM((_TPW,), jnp.int32),
            pltpu.VMEM((_TPW, D), F32),
            pltpu.SemaphoreType.DMA,
        ],
    )
    def k(x_hbm, i_hbm, o_hbm, idx_v, rows_v, sem):
        wid = jax.lax.axis_index("s") * _NC + jax.lax.axis_index("c")
        base = wid * _TPW
        pltpu.sync_copy(i_hbm.at[pl.ds(base, _TPW)], idx_v)
        pltpu.sync_copy(x_hbm.at[pl.ds(base, _TPW)], rows_v)
        pltpu.async_copy(rows_v, o_hbm.at[idx_v], sem).wait()

    return k(xf, slots)


def _sc_combine(ob, slots):
    @functools.partial(
        pl.kernel,
        mesh=_sc_mesh(),
        out_type=jax.ShapeDtypeStruct((BS, D), F32),
        scratch_types=[
            pltpu.VMEM((_TPW,), jnp.int32),
            pltpu.VMEM((_TPW, D), F32),
            pltpu.SemaphoreType.DMA,
        ],
    )
    def k(t_hbm, i_hbm, o_hbm, idx_v, rows_v, sem):
        wid = jax.lax.axis_index("s") * _NC + jax.lax.axis_index("c")
        base = wid * _TPW
        pltpu.sync_copy(i_hbm.at[pl.ds(base, _TPW)], idx_v)
        pltpu.async_copy(t_hbm.at[idx_v], rows_v, sem).wait()
        pltpu.sync_copy(rows_v, o_hbm.at[pl.ds(base, _TPW)])

    return k(ob, slots)


def kernel(x, ln1_g, ln1_b, ln2_g, ln2_b, conv_w, Wq, bq, Wk, bk, Wv, bv,
           Wo, bo, router_w, W1, b1, W2, b2):
    ln1g = ln1_g.reshape(1, D)
    ln1b = ln1_b.reshape(1, D)
    ln2g = ln2_g.reshape(1, D)
    ln2b = ln2_b.reshape(1, D)
    bq2 = bq.reshape(1, D)
    bk2 = bk.reshape(1, D)
    bv2 = bv.reshape(1, D)
    bo2 = bo.reshape(1, D)
    b1r = b1.reshape(E, 1, DFF)
    b2r = b2.reshape(E, 1, D)
    aux = jnp.zeros((), F32)
    for li in range(2):
        if li == 0:
            xr, h = _conv_ln(x, conv_w, ln1g, ln1b)
        else:
            xr, h = _comb_conv_ln(x2, gat.reshape(B, S, D),
                                  kg.reshape(B, S, 1), conv_w, ln1g, ln1b)
        o = _qkv_attn(h, Wq, bq2, Wk, bk2, Wv, bv2)
        x2, h2, lg = _proj_ln(o, Wo, bo2, xr, ln2g, ln2b, router_w)
        h2f = h2.reshape(BS, D)
        ss, sg, kg, lbarr = _router(lg.reshape(BS, E))
        buf = _sc_dispatch(h2f, ss.reshape(BS))
        ob = _expert_ffn(buf, W1, b1r, W2, b2r)
        gat = _sc_combine(ob, sg.reshape(BS))
        aux = aux + LBW * lbarr[0, 0]
    x3f = _combine(x2.reshape(BS, D), gat, kg)
    return x3f.reshape(B, S, D), aux


# speedup vs baseline: 1.2159x; 1.0904x over previous
import functools

import jax
import jax.numpy as jnp
import numpy as np
from jax.experimental import pallas as pl
from jax.experimental.pallas import tpu as pltpu
from jax.experimental.pallas import tpu_sc as plsc

B = 2; S = 2048; D = 768; H = 12; DH = D // H
POOL = 8; SP = S // POOL
E = 16; DFF = 2 * D
BS = B * S
CAP = int(np.ceil(1.25 * BS / E))
NSLOT = E * CAP
NSLOT_PAD = NSLOT + 8
LBW = 0.01
ATT_SCALE = float(1.0 / np.sqrt(DH))
F32 = jnp.float32
BF16 = jnp.bfloat16


def _conv_ln_core(x, w_ref, g_ref, b_ref, xr_ref, h_ref):
    z = jnp.zeros((1, D), F32)
    w = w_ref[...]
    xp = jnp.concatenate([z, x[:-1, :]], axis=0)
    xn = jnp.concatenate([x[1:, :], z], axis=0)
    xr = x + xp * w[0:1, :] + x * w[1:2, :] + xn * w[2:3, :]
    xr_ref[0] = xr
    mu = jnp.mean(xr, axis=-1, keepdims=True)
    var = jnp.mean((xr - mu) ** 2, axis=-1, keepdims=True)
    h = (xr - mu) * jax.lax.rsqrt(var + 1e-5) * g_ref[...] + b_ref[...]
    h_ref[0] = h.astype(BF16)


def _conv_ln_body(x_ref, w_ref, g_ref, b_ref, xr_ref, h_ref):
    _conv_ln_core(x_ref[0], w_ref, g_ref, b_ref, xr_ref, h_ref)


def _conv_ln(x, conv_w, g, b):
    return pl.pallas_call(
        _conv_ln_body,
        grid=(B,),
        in_specs=[
            pl.BlockSpec((1, S, D), lambda i: (i, 0, 0)),
            pl.BlockSpec((3, D), lambda i: (0, 0)),
            pl.BlockSpec((1, D), lambda i: (0, 0)),
            pl.BlockSpec((1, D), lambda i: (0, 0)),
        ],
        out_specs=[
            pl.BlockSpec((1, S, D), lambda i: (i, 0, 0)),
            pl.BlockSpec((1, S, D), lambda i: (i, 0, 0)),
        ],
        out_shape=[jax.ShapeDtypeStruct((B, S, D), F32),
                   jax.ShapeDtypeStruct((B, S, D), BF16)],
    )(x, conv_w, g, b)


def _comb_conv_ln_body(x2_ref, gat_ref, kg_ref, w_ref, g_ref, b_ref,
                       xr_ref, h_ref):
    x = x2_ref[0] + gat_ref[0].astype(F32) * kg_ref[0]
    _conv_ln_core(x, w_ref, g_ref, b_ref, xr_ref, h_ref)


def _comb_conv_ln(x2, gat, kg, conv_w, g, b):
    return pl.pallas_call(
        _comb_conv_ln_body,
        grid=(B,),
        in_specs=[
            pl.BlockSpec((1, S, D), lambda i: (i, 0, 0)),
            pl.BlockSpec((1, S, D), lambda i: (i, 0, 0)),
            pl.BlockSpec((1, S, 1), lambda i: (i, 0, 0)),
            pl.BlockSpec((3, D), lambda i: (0, 0)),
            pl.BlockSpec((1, D), lambda i: (0, 0)),
            pl.BlockSpec((1, D), lambda i: (0, 0)),
        ],
        out_specs=[
            pl.BlockSpec((1, S, D), lambda i: (i, 0, 0)),
            pl.BlockSpec((1, S, D), lambda i: (i, 0, 0)),
        ],
        out_shape=[jax.ShapeDtypeStruct((B, S, D), F32),
                   jax.ShapeDtypeStruct((B, S, D), BF16)],
    )(x2, gat, kg, conv_w, g, b)


def _qkv_attn_body(h_ref, wq_ref, bq_ref, wk_ref, bk_ref, wv_ref, bv_ref,
                   o_ref):
    hb = h_ref[0]
    gi = jax.lax.broadcasted_iota(jnp.int32, (SP, S), 1)
    ci = jax.lax.broadcasted_iota(jnp.int32, (SP, S), 0)
    pm = jnp.where(gi // POOL == ci, 1.0 / POOL, 0.0).astype(BF16)
    hpb = jnp.dot(pm, hb, preferred_element_type=F32).astype(BF16)
    q = (jnp.dot(hb, wq_ref[...].astype(BF16), preferred_element_type=F32)
         + bq_ref[...]).astype(BF16)
    k = (jnp.dot(hpb, wk_ref[...].astype(BF16), preferred_element_type=F32)
         + bk_ref[...]).astype(BF16)
    v = (jnp.dot(hpb, wv_ref[...].astype(BF16), preferred_element_type=F32)
         + bv_ref[...]).astype(BF16)
    ones = jnp.ones((SP, DH), BF16)
    for j in range(H):
        qh = q[:, j * DH:(j + 1) * DH]
        kh = k[:, j * DH:(j + 1) * DH]
        vh = v[:, j * DH:(j + 1) * DH]
        s = jax.lax.dot_general(qh, kh, (((1,), (1,)), ((), ())),
                                preferred_element_type=F32) * ATT_SCALE
        eb = jnp.exp(s).astype(BF16)
        va = jnp.concatenate([vh, ones], axis=-1)
        ov = jnp.dot(eb, va, preferred_element_type=F32)
        oj = ov[:, :DH] * (1.0 / ov[:, DH:DH + 1])
        o_ref[0, :, j * DH:(j + 1) * DH] = oj.astype(BF16)


def _qkv_attn(h, wq, bq, wk, bk, wv, bv):
    wspec = pl.BlockSpec((D, D), lambda i: (0, 0))
    bspec = pl.BlockSpec((1, D), lambda i: (0, 0))
    return pl.pallas_call(
        _qkv_attn_body,
        grid=(B,),
        in_specs=[pl.BlockSpec((1, S, D), lambda i: (i, 0, 0)),
                  wspec, bspec, wspec, bspec, wspec, bspec],
        out_specs=pl.BlockSpec((1, S, D), lambda i: (i, 0, 0)),
        out_shape=jax.ShapeDtypeStruct((B, S, D), BF16),
    )(h, wq, bq, wk, bk, wv, bv)


def _proj_ln_body(o_ref, wo_ref, bo_ref, xr_ref, g_ref, b_ref, rw_ref,
                  x2_ref, h2_ref, lg_ref):
    x2 = xr_ref[0] + jnp.dot(o_ref[0], wo_ref[...].astype(BF16),
                             preferred_element_type=F32) + bo_ref[...]
    x2_ref[0] = x2
    mu = jnp.mean(x2, axis=-1, keepdims=True)
    var = jnp.mean((x2 - mu) ** 2, axis=-1, keepdims=True)
    h2 = (x2 - mu) * jax.lax.rsqrt(var + 1e-5) * g_ref[...] + b_ref[...]
    h2_ref[0] = h2
    lg_ref[0] = jnp.dot(h2, rw_ref[...], preferred_element_type=F32)


_PBLK = S // 2


def _proj_ln(o, wo, bo, xr, g, b, rw):
    return pl.pallas_call(
        _proj_ln_body,
        grid=(B, S // _PBLK),
        in_specs=[
            pl.BlockSpec((1, _PBLK, D), lambda i, j: (i, j, 0)),
            pl.BlockSpec((D, D), lambda i, j: (0, 0)),
            pl.BlockSpec((1, D), lambda i, j: (0, 0)),
            pl.BlockSpec((1, _PBLK, D), lambda i, j: (i, j, 0)),
            pl.BlockSpec((1, D), lambda i, j: (0, 0)),
            pl.BlockSpec((1, D), lambda i, j: (0, 0)),
            pl.BlockSpec((D, E), lambda i, j: (0, 0)),
        ],
        out_specs=[
            pl.BlockSpec((1, _PBLK, D), lambda i, j: (i, j, 0)),
            pl.BlockSpec((1, _PBLK, D), lambda i, j: (i, j, 0)),
            pl.BlockSpec((1, _PBLK, E), lambda i, j: (i, j, 0)),
        ],
        out_shape=[jax.ShapeDtypeStruct((B, S, D), F32),
                   jax.ShapeDtypeStruct((B, S, D), F32),
                   jax.ShapeDtypeStruct((B, S, E), F32)],
    )(o, wo, bo, xr, g, b, rw)


_RCH = 512


def _router_body(lg_ref, ss_ref, sg_ref, kg_ref, lb_ref, oh_scr):
    logits = lg_ref[...]
    m = jnp.max(logits, axis=-1, keepdims=True)
    ex = jnp.exp(logits - m)
    probs = ex / jnp.sum(ex, axis=-1, keepdims=True)
    gate = jnp.max(probs, axis=-1, keepdims=True)
    ii = jax.lax.broadcasted_iota(jnp.int32, (BS, E), 1)
    eidx = jnp.min(jnp.where(probs == gate, ii, E), axis=-1, keepdims=True)
    oh = (ii == eidx).astype(F32)
    oh_scr[...] = oh
    pi = jnp.mean(probs, axis=0, keepdims=True)
    fi = jnp.mean(oh, axis=0, keepdims=True)
    lb = E * jnp.sum(fi * pi)
    lb_ref[...] = jnp.full((8, 128), lb, F32)
    sg_ref[...] = eidx
    kg_ref[...] = gate
    li = jax.lax.broadcasted_iota(jnp.int32, (_RCH, _RCH), 0)
    lj = jax.lax.broadcasted_iota(jnp.int32, (_RCH, _RCH), 1)
    ltri = jnp.where(lj <= li, 1.0, 0.0).astype(F32)

    def body(c, carry):
        sl = pl.ds(c * _RCH, _RCH)
        ohc = oh_scr[sl, :]
        within = jnp.dot(ltri, ohc, preferred_element_type=F32)
        pos = jnp.sum((within + carry) * ohc, axis=-1, keepdims=True) - 1.0
        e_c = sg_ref[sl, :]
        g_c = kg_ref[sl, :]
        keep = pos < CAP
        spos = jnp.clip(pos, 0.0, CAP - 1.0).astype(jnp.int32)
        slotg = e_c * CAP + spos
        ss_ref[sl, :] = jnp.where(keep, slotg, NSLOT)
        sg_ref[sl, :] = slotg
        kg_ref[sl, :] = jnp.where(keep, g_c, 0.0)
        return carry + within[_RCH - 1:_RCH, :]

    jax.lax.fori_loop(0, BS // _RCH, body, jnp.zeros((1, E), F32))


def _router(lg):
    return pl.pallas_call(
        _router_body,
        in_specs=[pl.BlockSpec((BS, E), lambda: (0, 0))],
        out_specs=[
            pl.BlockSpec((BS, 1), lambda: (0, 0)),
            pl.BlockSpec((BS, 1), lambda: (0, 0)),
            pl.BlockSpec((BS, 1), lambda: (0, 0)),
            pl.BlockSpec((8, 128), lambda: (0, 0)),
        ],
        out_shape=[
            jax.ShapeDtypeStruct((BS, 1), jnp.int32),
            jax.ShapeDtypeStruct((BS, 1), jnp.int32),
            jax.ShapeDtypeStruct((BS, 1), F32),
            jax.ShapeDtypeStruct((8, 128), F32),
        ],
        scratch_shapes=[pltpu.VMEM((BS, E), F32)],
    )(lg)


def _expert_body(buf_ref, w1_ref, b1_ref, w2_ref, b2_ref, ob_ref):
    xb = buf_ref[...].astype(BF16)
    hdn = jnp.dot(xb, w1_ref[0].astype(BF16),
                  preferred_element_type=F32) + b1_ref[0]
    hdn = jax.nn.gelu(hdn).astype(BF16)
    ob_ref[...] = jnp.dot(hdn, w2_ref[0].astype(BF16),
                          preferred_element_type=F32) + b2_ref[0]


def _expert_ffn(buf, w1, b1r, w2, b2r):
    return pl.pallas_call(
        _expert_body,
        grid=(E,),
        in_specs=[
            pl.BlockSpec((CAP, D), lambda i: (i, 0)),
            pl.BlockSpec((1, D, DFF), lambda i: (i, 0, 0)),
            pl.BlockSpec((1, 1, DFF), lambda i: (i, 0, 0)),
            pl.BlockSpec((1, DFF, D), lambda i: (i, 0, 0)),
            pl.BlockSpec((1, 1, D), lambda i: (i, 0, 0)),
        ],
        out_specs=pl.BlockSpec((CAP, D), lambda i: (i, 0)),
        out_shape=jax.ShapeDtypeStruct((NSLOT, D), F32),
    )(buf, w1, b1r, w2, b2r)


_CCH = 512


def _combine_body(x2_ref, g_ref, kg_ref, x3_ref):
    x3_ref[...] = x2_ref[...] + g_ref[...].astype(F32) * kg_ref[...]


def _combine(x2f, gat, kg):
    return pl.pallas_call(
        _combine_body,
        grid=(BS // _CCH,),
        in_specs=[
            pl.BlockSpec((_CCH, D), lambda i: (i, 0)),
            pl.BlockSpec((_CCH, D), lambda i: (i, 0)),
            pl.BlockSpec((_CCH, 1), lambda i: (i, 0)),
        ],
        out_specs=pl.BlockSpec((_CCH, D), lambda i: (i, 0)),
        out_shape=jax.ShapeDtypeStruct((BS, D), F32),
    )(x2f, gat, kg)


_NC = 2
_NS = 16
_NW = _NC * _NS
_TPW = BS // _NW


def _sc_mesh():
    return plsc.VectorSubcoreMesh(core_axis_name="c", subcore_axis_name="s")


def _sc_dispatch(xf, slots):
    @functools.partial(
        pl.kernel,
        mesh=_sc_mesh(),
        out_type=jax.ShapeDtypeStruct((NSLOT_PAD, D), F32),
        scratch_types=[
            pltpu.VMEM((_TPW,), jnp.int32),
            pltpu.VMEM((_TPW, D), F32),
            pltpu.SemaphoreType.DMA,
        ],
    )
    def k(x_hbm, i_hbm, o_hbm, idx_v, rows_v, sem):
        wid = jax.lax.axis_index("s") * _NC + jax.lax.axis_index("c")
        base = wid * _TPW
        pltpu.sync_copy(i_hbm.at[pl.ds(base, _TPW)], idx_v)
        pltpu.sync_copy(x_hbm.at[pl.ds(base, _TPW)], rows_v)
        pltpu.async_copy(rows_v, o_hbm.at[idx_v], sem).wait()

    return k(xf, slots)


def _sc_combine(ob, slots):
    @functools.partial(
        pl.kernel,
        mesh=_sc_mesh(),
        out_type=jax.ShapeDtypeStruct((BS, D), F32),
        scratch_types=[
            pltpu.VMEM((_TPW,), jnp.int32),
            pltpu.VMEM((_TPW, D), F32),
            pltpu.SemaphoreType.DMA,
        ],
    )
    def k(t_hbm, i_hbm, o_hbm, idx_v, rows_v, sem):
        wid = jax.lax.axis_index("s") * _NC + jax.lax.axis_index("c")
        base = wid * _TPW
        pltpu.sync_copy(i_hbm.at[pl.ds(base, _TPW)], idx_v)
        pltpu.async_copy(t_hbm.at[idx_v], rows_v, sem).wait()
        pltpu.sync_copy(rows_v, o_hbm.at[pl.ds(base, _TPW)])

    return k(ob, slots)


def kernel(x, ln1_g, ln1_b, ln2_g, ln2_b, conv_w, Wq, bq, Wk, bk, Wv, bv,
           Wo, bo, router_w, W1, b1, W2, b2):
    ln1g = ln1_g.reshape(1, D)
    ln1b = ln1_b.reshape(1, D)
    ln2g = ln2_g.reshape(1, D)
    ln2b = ln2_b.reshape(1, D)
    bq2 = bq.reshape(1, D)
    bk2 = bk.reshape(1, D)
    bv2 = bv.reshape(1, D)
    bo2 = bo.reshape(1, D)
    b1r = b1.reshape(E, 1, DFF)
    b2r = b2.reshape(E, 1, D)
    aux = jnp.zeros((), F32)
    for li in range(2):
        if li == 0:
            xr, h = _conv_ln(x, conv_w, ln1g, ln1b)
        else:
            xr, h = _comb_conv_ln(x2, gat.reshape(B, S, D),
                                  kg.reshape(B, S, 1), conv_w, ln1g, ln1b)
        o = _qkv_attn(h, Wq, bq2, Wk, bk2, Wv, bv2)
        x2, h2, lg = _proj_ln(o, Wo, bo2, xr, ln2g, ln2b, router_w)
        h2f = h2.reshape(BS, D)
        ss, sg, kg, lbarr = _router(lg.reshape(BS, E))
        buf = _sc_dispatch(h2f, ss.reshape(BS))
        ob = _expert_ffn(buf, W1, b1r, W2, b2r)
        gat = _sc_combine(ob, sg.reshape(BS))
        aux = aux + LBW * lbarr[0, 0]
    x3f = _combine(x2.reshape(BS, D), gat, kg)
    return x3f.reshape(B, S, D), aux


# speedup vs baseline: 1.2320x; 1.0132x over previous
import functools

import jax
import jax.numpy as jnp
import numpy as np
from jax.experimental import pallas as pl
from jax.experimental.pallas import tpu as pltpu
from jax.experimental.pallas import tpu_sc as plsc

B = 2; S = 2048; D = 768; H = 12; DH = D // H
POOL = 8; SP = S // POOL
E = 16; DFF = 2 * D
BS = B * S
CAP = int(np.ceil(1.25 * BS / E))
NSLOT = E * CAP
NSLOT_PAD = NSLOT + 8
LBW = 0.01
ATT_SCALE = float(1.0 / np.sqrt(DH))
F32 = jnp.float32
BF16 = jnp.bfloat16


def _conv_ln_core(x, w_ref, g_ref, b_ref, xr_ref, h_ref):
    z = jnp.zeros((1, D), F32)
    w = w_ref[...]
    xp = jnp.concatenate([z, x[:-1, :]], axis=0)
    xn = jnp.concatenate([x[1:, :], z], axis=0)
    xr = x + xp * w[0:1, :] + x * w[1:2, :] + xn * w[2:3, :]
    xr_ref[0] = xr
    mu = jnp.mean(xr, axis=-1, keepdims=True)
    var = jnp.mean((xr - mu) ** 2, axis=-1, keepdims=True)
    h = (xr - mu) * jax.lax.rsqrt(var + 1e-5) * g_ref[...] + b_ref[...]
    h_ref[0] = h.astype(BF16)


def _conv_ln_body(x_ref, w_ref, g_ref, b_ref, xr_ref, h_ref):
    _conv_ln_core(x_ref[0], w_ref, g_ref, b_ref, xr_ref, h_ref)


def _conv_ln(x, conv_w, g, b):
    return pl.pallas_call(
        _conv_ln_body,
        grid=(B,),
        in_specs=[
            pl.BlockSpec((1, S, D), lambda i: (i, 0, 0)),
            pl.BlockSpec((3, D), lambda i: (0, 0)),
            pl.BlockSpec((1, D), lambda i: (0, 0)),
            pl.BlockSpec((1, D), lambda i: (0, 0)),
        ],
        out_specs=[
            pl.BlockSpec((1, S, D), lambda i: (i, 0, 0)),
            pl.BlockSpec((1, S, D), lambda i: (i, 0, 0)),
        ],
        out_shape=[jax.ShapeDtypeStruct((B, S, D), F32),
                   jax.ShapeDtypeStruct((B, S, D), BF16)],
    )(x, conv_w, g, b)


def _comb_conv_ln_body(x2_ref, gat_ref, kg_ref, w_ref, g_ref, b_ref,
                       xr_ref, h_ref):
    x = x2_ref[0] + gat_ref[0].astype(F32) * kg_ref[0]
    _conv_ln_core(x, w_ref, g_ref, b_ref, xr_ref, h_ref)


def _comb_conv_ln(x2, gat, kg, conv_w, g, b):
    return pl.pallas_call(
        _comb_conv_ln_body,
        grid=(B,),
        in_specs=[
            pl.BlockSpec((1, S, D), lambda i: (i, 0, 0)),
            pl.BlockSpec((1, S, D), lambda i: (i, 0, 0)),
            pl.BlockSpec((1, S, 1), lambda i: (i, 0, 0)),
            pl.BlockSpec((3, D), lambda i: (0, 0)),
            pl.BlockSpec((1, D), lambda i: (0, 0)),
            pl.BlockSpec((1, D), lambda i: (0, 0)),
        ],
        out_specs=[
            pl.BlockSpec((1, S, D), lambda i: (i, 0, 0)),
            pl.BlockSpec((1, S, D), lambda i: (i, 0, 0)),
        ],
        out_shape=[jax.ShapeDtypeStruct((B, S, D), F32),
                   jax.ShapeDtypeStruct((B, S, D), BF16)],
    )(x2, gat, kg, conv_w, g, b)


def _qkv_attn_body(h_ref, wq_ref, bq_ref, wk_ref, bk_ref, wv_ref, bv_ref,
                   o_ref):
    hb = h_ref[0]
    gi = jax.lax.broadcasted_iota(jnp.int32, (SP, S), 1)
    ci = jax.lax.broadcasted_iota(jnp.int32, (SP, S), 0)
    pm = jnp.where(gi // POOL == ci, 1.0 / POOL, 0.0).astype(BF16)
    hpb = jnp.dot(pm, hb, preferred_element_type=F32).astype(BF16)
    q = (jnp.dot(hb, wq_ref[...].astype(BF16), preferred_element_type=F32)
         + bq_ref[...]).astype(BF16)
    k = (jnp.dot(hpb, wk_ref[...].astype(BF16), preferred_element_type=F32)
         + bk_ref[...]).astype(BF16)
    v = (jnp.dot(hpb, wv_ref[...].astype(BF16), preferred_element_type=F32)
         + bv_ref[...]).astype(BF16)
    ones = jnp.ones((SP, DH), BF16)
    for j in range(H):
        qh = q[:, j * DH:(j + 1) * DH]
        kh = k[:, j * DH:(j + 1) * DH]
        vh = v[:, j * DH:(j + 1) * DH]
        s = jax.lax.dot_general(qh, kh, (((1,), (1,)), ((), ())),
                                preferred_element_type=F32) * ATT_SCALE
        eb = jnp.exp(s).astype(BF16)
        va = jnp.concatenate([vh, ones], axis=-1)
        ov = jnp.dot(eb, va, preferred_element_type=F32)
        oj = ov[:, :DH] * (1.0 / ov[:, DH:DH + 1])
        o_ref[0, :, j * DH:(j + 1) * DH] = oj.astype(BF16)


def _qkv_attn(h, wq, bq, wk, bk, wv, bv):
    wspec = pl.BlockSpec((D, D), lambda i: (0, 0))
    bspec = pl.BlockSpec((1, D), lambda i: (0, 0))
    return pl.pallas_call(
        _qkv_attn_body,
        grid=(B,),
        in_specs=[pl.BlockSpec((1, S, D), lambda i: (i, 0, 0)),
                  wspec, bspec, wspec, bspec, wspec, bspec],
        out_specs=pl.BlockSpec((1, S, D), lambda i: (i, 0, 0)),
        out_shape=jax.ShapeDtypeStruct((B, S, D), BF16),
    )(h, wq, bq, wk, bk, wv, bv)


_RCH = 512
_PBLK = S // 2


def _proj_ln_route_body(o_ref, wo_ref, bo_ref, xr_ref, g_ref, b_ref, rw_ref,
                        x2_ref, h2_ref, ss_ref, sg_ref, kg_ref, lb_ref,
                        lg_scr, oh_scr):
    bi = pl.program_id(0)
    j = pl.program_id(1)
    x2 = xr_ref[0] + jnp.dot(o_ref[0], wo_ref[...].astype(BF16),
                             preferred_element_type=F32) + bo_ref[...]
    x2_ref[0] = x2
    mu = jnp.mean(x2, axis=-1, keepdims=True)
    var = jnp.mean((x2 - mu) ** 2, axis=-1, keepdims=True)
    h2 = (x2 - mu) * jax.lax.rsqrt(var + 1e-5) * g_ref[...] + b_ref[...]
    h2_ref[0] = h2
    base = bi * S + j * _PBLK
    lg_scr[pl.ds(base, _PBLK), :] = jnp.dot(h2, rw_ref[...],
                                            preferred_element_type=F32)

    @pl.when((bi == B - 1) & (j == S // _PBLK - 1))
    def _route():
        logits = lg_scr[...]
        m = jnp.max(logits, axis=-1, keepdims=True)
        ex = jnp.exp(logits - m)
        probs = ex / jnp.sum(ex, axis=-1, keepdims=True)
        gate = jnp.max(probs, axis=-1, keepdims=True)
        ii = jax.lax.broadcasted_iota(jnp.int32, (BS, E), 1)
        eidx = jnp.min(jnp.where(probs == gate, ii, E), axis=-1, keepdims=True)
        oh = (ii == eidx).astype(F32)
        oh_scr[...] = oh
        pi = jnp.mean(probs, axis=0, keepdims=True)
        fi = jnp.mean(oh, axis=0, keepdims=True)
        lb = E * jnp.sum(fi * pi)
        lb_ref[...] = jnp.full((8, 128), lb, F32)
        sg_ref[...] = eidx
        kg_ref[...] = gate
        li = jax.lax.broadcasted_iota(jnp.int32, (_RCH, _RCH), 0)
        lj = jax.lax.broadcasted_iota(jnp.int32, (_RCH, _RCH), 1)
        ltri = jnp.where(lj <= li, 1.0, 0.0).astype(F32)

        def body(c, carry):
            sl = pl.ds(c * _RCH, _RCH)
            ohc = oh_scr[sl, :]
            within = jnp.dot(ltri, ohc, preferred_element_type=F32)
            pos = jnp.sum((within + carry) * ohc, axis=-1, keepdims=True) - 1.0
            e_c = sg_ref[sl, :]
            g_c = kg_ref[sl, :]
            keep = pos < CAP
            spos = jnp.clip(pos, 0.0, CAP - 1.0).astype(jnp.int32)
            slotg = e_c * CAP + spos
            ss_ref[sl, :] = jnp.where(keep, slotg, NSLOT)
            sg_ref[sl, :] = slotg
            kg_ref[sl, :] = jnp.where(keep, g_c, 0.0)
            return carry + within[_RCH - 1:_RCH, :]

        jax.lax.fori_loop(0, BS // _RCH, body, jnp.zeros((1, E), F32))


def _proj_ln_route(o, wo, bo, xr, g, b, rw):
    full = lambda i, j: (0, 0)
    return pl.pallas_call(
        _proj_ln_route_body,
        grid=(B, S // _PBLK),
        in_specs=[
            pl.BlockSpec((1, _PBLK, D), lambda i, j: (i, j, 0)),
            pl.BlockSpec((D, D), full),
            pl.BlockSpec((1, D), full),
            pl.BlockSpec((1, _PBLK, D), lambda i, j: (i, j, 0)),
            pl.BlockSpec((1, D), full),
            pl.BlockSpec((1, D), full),
            pl.BlockSpec((D, E), full),
        ],
        out_specs=[
            pl.BlockSpec((1, _PBLK, D), lambda i, j: (i, j, 0)),
            pl.BlockSpec((1, _PBLK, D), lambda i, j: (i, j, 0)),
            pl.BlockSpec((BS, 1), full),
            pl.BlockSpec((BS, 1), full),
            pl.BlockSpec((BS, 1), full),
            pl.BlockSpec((8, 128), full),
        ],
        out_shape=[
            jax.ShapeDtypeStruct((B, S, D), F32),
            jax.ShapeDtypeStruct((B, S, D), F32),
            jax.ShapeDtypeStruct((BS, 1), jnp.int32),
            jax.ShapeDtypeStruct((BS, 1), jnp.int32),
            jax.ShapeDtypeStruct((BS, 1), F32),
            jax.ShapeDtypeStruct((8, 128), F32),
        ],
        scratch_shapes=[pltpu.VMEM((BS, E), F32), pltpu.VMEM((BS, E), F32)],
    )(o, wo, bo, xr, g, b, rw)


def _expert_body(buf_ref, w1_ref, b1_ref, w2_ref, b2_ref, ob_ref):
    xb = buf_ref[...].astype(BF16)
    hdn = jnp.dot(xb, w1_ref[0].astype(BF16),
                  preferred_element_type=F32) + b1_ref[0]
    hdn = jax.nn.gelu(hdn).astype(BF16)
    ob_ref[...] = jnp.dot(hdn, w2_ref[0].astype(BF16),
                          preferred_element_type=F32) + b2_ref[0]


def _expert_ffn(buf, w1, b1r, w2, b2r):
    return pl.pallas_call(
        _expert_body,
        grid=(E,),
        in_specs=[
            pl.BlockSpec((CAP, D), lambda i: (i, 0)),
            pl.BlockSpec((1, D, DFF), lambda i: (i, 0, 0)),
            pl.BlockSpec((1, 1, DFF), lambda i: (i, 0, 0)),
            pl.BlockSpec((1, DFF, D), lambda i: (i, 0, 0)),
            pl.BlockSpec((1, 1, D), lambda i: (i, 0, 0)),
        ],
        out_specs=pl.BlockSpec((CAP, D), lambda i: (i, 0)),
        out_shape=jax.ShapeDtypeStruct((NSLOT, D), F32),
    )(buf, w1, b1r, w2, b2r)


_CCH = 512


def _combine_body(x2_ref, g_ref, kg_ref, x3_ref):
    x3_ref[...] = x2_ref[...] + g_ref[...].astype(F32) * kg_ref[...]


def _combine(x2f, gat, kg):
    return pl.pallas_call(
        _combine_body,
        grid=(BS // _CCH,),
        in_specs=[
            pl.BlockSpec((_CCH, D), lambda i: (i, 0)),
            pl.BlockSpec((_CCH, D), lambda i: (i, 0)),
            pl.BlockSpec((_CCH, 1), lambda i: (i, 0)),
        ],
        out_specs=pl.BlockSpec((_CCH, D), lambda i: (i, 0)),
        out_shape=jax.ShapeDtypeStruct((BS, D), F32),
    )(x2f, gat, kg)


_NC = 2
_NS = 16
_NW = _NC * _NS
_TPW = BS // _NW


def _sc_mesh():
    return plsc.VectorSubcoreMesh(core_axis_name="c", subcore_axis_name="s")


def _sc_dispatch(xf, slots):
    @functools.partial(
        pl.kernel,
        mesh=_sc_mesh(),
        out_type=jax.ShapeDtypeStruct((NSLOT_PAD, D), F32),
        scratch_types=[
            pltpu.VMEM((_TPW,), jnp.int32),
            pltpu.VMEM((_TPW, D), F32),
            pltpu.SemaphoreType.DMA,
        ],
    )
    def k(x_hbm, i_hbm, o_hbm, idx_v, rows_v, sem):
        wid = jax.lax.axis_index("s") * _NC + jax.lax.axis_index("c")
        base = wid * _TPW
        pltpu.sync_copy(i_hbm.at[pl.ds(base, _TPW)], idx_v)
        pltpu.sync_copy(x_hbm.at[pl.ds(base, _TPW)], rows_v)
        pltpu.async_copy(rows_v, o_hbm.at[idx_v], sem).wait()

    return k(xf, slots)


def _sc_combine(ob, slots):
    @functools.partial(
        pl.kernel,
        mesh=_sc_mesh(),
        out_type=jax.ShapeDtypeStruct((BS, D), F32),
        scratch_types=[
            pltpu.VMEM((_TPW,), jnp.int32),
            pltpu.VMEM((_TPW, D), F32),
            pltpu.SemaphoreType.DMA,
        ],
    )
    def k(t_hbm, i_hbm, o_hbm, idx_v, rows_v, sem):
        wid = jax.lax.axis_index("s") * _NC + jax.lax.axis_index("c")
        base = wid * _TPW
        pltpu.sync_copy(i_hbm.at[pl.ds(base, _TPW)], idx_v)
        pltpu.async_copy(t_hbm.at[idx_v], rows_v, sem).wait()
        pltpu.sync_copy(rows_v, o_hbm.at[pl.ds(base, _TPW)])

    return k(ob, slots)


def kernel(x, ln1_g, ln1_b, ln2_g, ln2_b, conv_w, Wq, bq, Wk, bk, Wv, bv,
           Wo, bo, router_w, W1, b1, W2, b2):
    ln1g = ln1_g.reshape(1, D)
    ln1b = ln1_b.reshape(1, D)
    ln2g = ln2_g.reshape(1, D)
    ln2b = ln2_b.reshape(1, D)
    bq2 = bq.reshape(1, D)
    bk2 = bk.reshape(1, D)
    bv2 = bv.reshape(1, D)
    bo2 = bo.reshape(1, D)
    b1r = b1.reshape(E, 1, DFF)
    b2r = b2.reshape(E, 1, D)
    aux = jnp.zeros((), F32)
    for li in range(2):
        if li == 0:
            xr, h = _conv_ln(x, conv_w, ln1g, ln1b)
        else:
            xr, h = _comb_conv_ln(x2, gat.reshape(B, S, D),
                                  kg.reshape(B, S, 1), conv_w, ln1g, ln1b)
        o = _qkv_attn(h, Wq, bq2, Wk, bk2, Wv, bv2)
        x2, h2, ss, sg, kg, lbarr = _proj_ln_route(o, Wo, bo2, xr,
                                                   ln2g, ln2b, router_w)
        h2f = h2.reshape(BS, D)
        buf = _sc_dispatch(h2f, ss.reshape(BS))
        ob = _expert_ffn(buf, W1, b1r, W2, b2r)
        gat = _sc_combine(ob, sg.reshape(BS))
        aux = aux + LBW * lbarr[0, 0]
    x3f = _combine(x2.reshape(BS, D), gat, kg)
    return x3f.reshape(B, S, D), aux
